# Initial kernel scaffold; baseline (speedup 1.0000x reference)
import jax
import jax.numpy as jnp
from jax.experimental import pallas as pl


def kernel(x, edge_index, Wp0, bp0, Ws0, Wn0, b0, g0, be0, Wp1, bp1, Ws1, Wn1, b1, g1, be1, fc1_w, fc1_b, fc_w, fc_b):
    raise NotImplementedError("write your pallas kernel here")


# speedup vs baseline: 1.7420x; 1.7420x over previous
import functools

import jax
import jax.numpy as jnp
from jax import lax
from jax.experimental import pallas as pl
from jax.experimental.pallas import tpu as pltpu
from jax.experimental.pallas import tpu_sc as plsc

_NC = 2
_NS = 16
_NW = _NC * _NS
_L = 16

_CHUNK = 1280
_GB = 128


def _build_segment_max(n_edges, d, rows_per_w, npad):
    nchunk = n_edges // _CHUNK
    nf = d // _L
    cbuf = _CHUNK + 2 * _GB
    mesh = plsc.VectorSubcoreMesh(
        core_axis_name="c", subcore_axis_name="s",
        num_cores=_NC, num_subcores=_NS)

    @functools.partial(
        pl.kernel,
        mesh=mesh,
        compiler_params=pltpu.CompilerParams(
            needs_layout_passes=False, use_tc_tiling_on_sc=False),
        out_type=jax.ShapeDtypeStruct((npad, d), jnp.float32),
        scratch_types=[
            pltpu.VMEM((rows_per_w + 1, d), jnp.float32),
            pltpu.VMEM((_CHUNK,), jnp.int32),
            pltpu.VMEM((_CHUNK,), jnp.int32),
            pltpu.VMEM((cbuf,), jnp.int32),
            pltpu.VMEM((cbuf,), jnp.int32),
            pltpu.VMEM((_GB, d), jnp.float32),
            pltpu.SemaphoreType.DMA,
        ],
    )
    def seg_max(feat, src, dst, out, acc, sv, dv, cs, cd, rows, sem):
        wid = lax.axis_index("s") * _NC + lax.axis_index("c")
        lo = wid * rows_per_w
        zero16f = jnp.zeros((_L,), jnp.float32)

        def init_row(r, carry):
            for f in range(nf):
                acc[r, pl.ds(f * _L, _L)] = zero16f
            return carry
        lax.fori_loop(0, rows_per_w + 1, init_row, 0)

        def do_batches(nb):
            def batch(b, carry):
                pltpu.async_copy(
                    feat.at[cs.at[pl.ds(b * _GB, _GB)]], rows, sem).wait()

                def row_max(g, c3):
                    dv16 = cd[pl.ds(b * _GB + g * _L, _L)]
                    for k in range(_L):
                        r = dv16[k]
                        j = g * _L + k
                        for f in range(nf):
                            sl = pl.ds(f * _L, _L)
                            acc[r, sl] = jnp.maximum(acc[r, sl], rows[j, sl])
                    return c3
                lax.fori_loop(0, _GB // _L, row_max, 0)
                return carry
            lax.fori_loop(0, nb, batch, 0)

        def chunk_body(c, cnt):
            pltpu.sync_copy(src.at[pl.ds(c * _CHUNK, _CHUNK)], sv)
            pltpu.sync_copy(dst.at[pl.ds(c * _CHUNK, _CHUNK)], dv)

            def scan(i, cnt2):
                dd = dv[pl.ds(i * _L, _L)]
                ss = sv[pl.ds(i * _L, _L)]
                m = (dd >= lo) & (dd < lo + rows_per_w)
                plsc.store_compressed(cs.at[pl.ds(cnt2, _L)], ss, mask=m)
                plsc.store_compressed(cd.at[pl.ds(cnt2, _L)], dd - lo, mask=m)
                pc = plsc.all_reduce_population_count(m)
                return cnt2 + pc[0]
            cnt = lax.fori_loop(0, _CHUNK // _L, scan, cnt)

            nb = cnt // _GB
            do_batches(nb)
            rem = cnt - nb * _GB
            base = nb * _GB
            for k in range(_GB // _L):
                vs = cs[pl.ds(base + k * _L, _L)]
                vd = cd[pl.ds(base + k * _L, _L)]
                cs[pl.ds(k * _L, _L)] = vs
                cd[pl.ds(k * _L, _L)] = vd
            return rem

        rem = lax.fori_loop(0, nchunk, chunk_body, 0)

        zero16i = jnp.zeros((_L,), jnp.int32)
        dummy16 = jnp.full((_L,), rows_per_w, jnp.int32)
        for k in range(_GB // _L):
            cs[pl.ds(rem + k * _L, _L)] = zero16i
            cd[pl.ds(rem + k * _L, _L)] = dummy16
        do_batches((rem + _GB - 1) // _GB)

        pltpu.sync_copy(acc.at[pl.ds(0, rows_per_w)],
                        out.at[pl.ds(lo, rows_per_w)])

    return seg_max


def _segment_max(feat, src, dst, n_nodes):
    d = feat.shape[1]
    e = src.shape[0]
    rows_per_w = (-(-n_nodes // _NW) + 7) // 8 * 8
    npad = _NW * rows_per_w
    epad = -(-e // _CHUNK) * _CHUNK
    if epad != e:
        pad = epad - e
        src = jnp.concatenate([src, jnp.zeros((pad,), jnp.int32)])
        dst = jnp.concatenate([dst, jnp.full((pad,), 0x3FFFFFFF, jnp.int32)])
    fn = _build_segment_max(epad, d, rows_per_w, npad)
    return fn(feat, src, dst)[:n_nodes]


def _relu_mm(x, w, b, bm):
    n, din = x.shape
    dout = w.shape[1]

    def body(x_ref, w_ref, b_ref, o_ref):
        o_ref[...] = jnp.maximum(
            jnp.dot(x_ref[...], w_ref[...], preferred_element_type=jnp.float32)
            + b_ref[...], 0.0)

    return pl.pallas_call(
        body,
        grid=(n // bm,),
        in_specs=[pl.BlockSpec((bm, din), lambda i: (i, 0)),
                  pl.BlockSpec((din, dout), lambda i: (0, 0)),
                  pl.BlockSpec((1, dout), lambda i: (0, 0))],
        out_specs=pl.BlockSpec((bm, dout), lambda i: (i, 0)),
        out_shape=jax.ShapeDtypeStruct((n, dout), jnp.float32),
    )(x, w, b.reshape(1, -1))


def _ln_relu(t, g_ref, be_ref):
    mu = jnp.mean(t, axis=-1, keepdims=True)
    var = jnp.mean((t - mu) ** 2, axis=-1, keepdims=True)
    hn = (t - mu) / jnp.sqrt(var + 1e-5) * g_ref[...] + be_ref[...]
    return jnp.maximum(hn, 0.0)


def _mid_layer(x, agg, ws, wn, b, g, be, wp1, bp1, bm):
    n, din = x.shape
    h = ws.shape[1]

    def body(x_ref, a_ref, ws_ref, wn_ref, b_ref, g_ref, be_ref, wp_ref,
             bp_ref, h0_ref, m1_ref):
        t = (jnp.dot(x_ref[...], ws_ref[...], preferred_element_type=jnp.float32)
             + jnp.dot(a_ref[...], wn_ref[...], preferred_element_type=jnp.float32)
             + b_ref[...])
        h0 = _ln_relu(t, g_ref, be_ref)
        h0_ref[...] = h0
        m1_ref[...] = jnp.maximum(
            jnp.dot(h0, wp_ref[...], preferred_element_type=jnp.float32)
            + bp_ref[...], 0.0)

    return pl.pallas_call(
        body,
        grid=(n // bm,),
        in_specs=[pl.BlockSpec((bm, din), lambda i: (i, 0)),
                  pl.BlockSpec((bm, din), lambda i: (i, 0)),
                  pl.BlockSpec((din, h), lambda i: (0, 0)),
                  pl.BlockSpec((din, h), lambda i: (0, 0)),
                  pl.BlockSpec((1, h), lambda i: (0, 0)),
                  pl.BlockSpec((1, h), lambda i: (0, 0)),
                  pl.BlockSpec((1, h), lambda i: (0, 0)),
                  pl.BlockSpec((h, h), lambda i: (0, 0)),
                  pl.BlockSpec((1, h), lambda i: (0, 0))],
        out_specs=[pl.BlockSpec((bm, h), lambda i: (i, 0)),
                   pl.BlockSpec((bm, h), lambda i: (i, 0))],
        out_shape=[jax.ShapeDtypeStruct((n, h), jnp.float32),
                   jax.ShapeDtypeStruct((n, h), jnp.float32)],
    )(x, agg, ws, wn, b.reshape(1, -1), g.reshape(1, -1), be.reshape(1, -1),
      wp1, bp1.reshape(1, -1))


def _final_layer(h0, agg, ws, wn, b, g, be, fc1w, fc1b, fcw, fcb, bm):
    n, h = h0.shape
    grid = n // bm

    def body(h0_ref, a_ref, ws_ref, wn_ref, b_ref, g_ref, be_ref, w1_ref,
             b1_ref, w2_ref, b2_ref, o_ref, pooled):
        i = pl.program_id(0)
        t = (jnp.dot(h0_ref[...], ws_ref[...], preferred_element_type=jnp.float32)
             + jnp.dot(a_ref[...], wn_ref[...], preferred_element_type=jnp.float32)
             + b_ref[...])
        h1 = _ln_relu(t, g_ref, be_ref)
        bmax = jnp.max(h1, axis=0, keepdims=True)

        @pl.when(i == 0)
        def _():
            pooled[...] = bmax

        @pl.when(i > 0)
        def _():
            pooled[...] = jnp.maximum(pooled[...], bmax)

        @pl.when(i == grid - 1)
        def _():
            y = (jnp.dot(pooled[...], w1_ref[...],
                         preferred_element_type=jnp.float32) + b1_ref[...])
            o_ref[...] = (jnp.dot(y, w2_ref[...],
                                  preferred_element_type=jnp.float32)
                          + b2_ref[...])

    return pl.pallas_call(
        body,
        grid=(grid,),
        in_specs=[pl.BlockSpec((bm, h), lambda i: (i, 0)),
                  pl.BlockSpec((bm, h), lambda i: (i, 0)),
                  pl.BlockSpec((h, h), lambda i: (0, 0)),
                  pl.BlockSpec((h, h), lambda i: (0, 0)),
                  pl.BlockSpec((1, h), lambda i: (0, 0)),
                  pl.BlockSpec((1, h), lambda i: (0, 0)),
                  pl.BlockSpec((1, h), lambda i: (0, 0)),
                  pl.BlockSpec((h, h), lambda i: (0, 0)),
                  pl.BlockSpec((1, h), lambda i: (0, 0)),
                  pl.BlockSpec((h, 1), lambda i: (0, 0)),
                  pl.BlockSpec((1, 1), lambda i: (0, 0))],
        out_specs=pl.BlockSpec((1, 1), lambda i: (0, 0)),
        out_shape=jax.ShapeDtypeStruct((1, 1), jnp.float32),
        scratch_shapes=[pltpu.VMEM((1, h), jnp.float32)],
    )(h0, agg, ws, wn, b.reshape(1, -1), g.reshape(1, -1), be.reshape(1, -1),
      fc1w, fc1b.reshape(1, -1), fcw, fcb.reshape(1, -1))


def kernel(x, edge_index, Wp0, bp0, Ws0, Wn0, b0, g0, be0, Wp1, bp1, Ws1,
           Wn1, b1, g1, be1, fc1_w, fc1_b, fc_w, fc_b):
    n = x.shape[0]
    src = edge_index[0]
    dst = edge_index[1]
    bm = 1000 if n % 1000 == 0 else n

    m0 = _relu_mm(x, Wp0, bp0, bm)
    agg0 = _segment_max(m0, src, dst, n)
    h0, m1 = _mid_layer(x, agg0, Ws0, Wn0, b0, g0, be0, Wp1, bp1, bm)
    agg1 = _segment_max(m1, src, dst, n)
    return _final_layer(h0, agg1, Ws1, Wn1, b1, g1, be1, fc1_w, fc1_b,
                        fc_w, fc_b, bm)


# speedup vs baseline: 3.1909x; 1.8317x over previous
import functools

import jax
import jax.numpy as jnp
from jax import lax
from jax.experimental import pallas as pl
from jax.experimental.pallas import tpu as pltpu
from jax.experimental.pallas import tpu_sc as plsc

_NC = 2
_NS = 16
_NW = _NC * _NS
_L = 16

_CHUNK = 1280
_GB = 128
_EMIT = 1024
_CB = 3072

_SC_PARAMS = pltpu.CompilerParams(
    needs_layout_passes=False, use_tc_tiling_on_sc=False)


def _mesh():
    return plsc.VectorSubcoreMesh(
        core_axis_name="c", subcore_axis_name="s",
        num_cores=_NC, num_subcores=_NS)


def _build_compact(n_edges, rows_per_w, maxb):
    nchunk = n_edges // _CHUNK
    assert nchunk % 2 == 0

    @functools.partial(
        pl.kernel,
        mesh=_mesh(),
        compiler_params=_SC_PARAMS,
        out_type=(jax.ShapeDtypeStruct((_NW, maxb * _GB), jnp.int32),
                  jax.ShapeDtypeStruct((_NW, maxb * _GB), jnp.int32),
                  jax.ShapeDtypeStruct((_NW, 16), jnp.int32)),
        scratch_types=[
            pltpu.VMEM((_CHUNK,), jnp.int32),
            pltpu.VMEM((_CHUNK,), jnp.int32),
            pltpu.VMEM((_CHUNK,), jnp.int32),
            pltpu.VMEM((_CHUNK,), jnp.int32),
            pltpu.VMEM((_CB,), jnp.int32),
            pltpu.VMEM((_CB,), jnp.int32),
            pltpu.VMEM((16,), jnp.int32),
            pltpu.SemaphoreType.DMA,
            pltpu.SemaphoreType.DMA,
            pltpu.SemaphoreType.DMA,
            pltpu.SemaphoreType.DMA,
        ],
    )
    def compact(src, dst, cs_all, cd_all, cnts, sva, dva, svb, dvb, cs, cd,
                cstage, sa, da, sb, db):
        wid = lax.axis_index("s") * _NC + lax.axis_index("c")
        lo = wid * rows_per_w

        def scan_chunk(sv, dv, cnt):
            def scan(i, cnt2):
                dd = dv[pl.ds(i * _L, _L)]
                ss = sv[pl.ds(i * _L, _L)]
                m = (dd >= lo) & (dd < lo + rows_per_w)
                plsc.store_compressed(cs.at[pl.ds(cnt2, _L)], ss, mask=m)
                plsc.store_compressed(cd.at[pl.ds(cnt2, _L)], dd - lo, mask=m)
                pc = plsc.all_reduce_population_count(m)
                return cnt2 + pc[0]
            return lax.fori_loop(0, _CHUNK // _L, scan, cnt)

        def emit_blocks(cnt, nbt):
            ne = cnt // _EMIT

            def emit(i, nbt2):
                pltpu.sync_copy(
                    cs.at[pl.ds(i * _EMIT, _EMIT)],
                    cs_all.at[wid, pl.ds(nbt2 * _GB, _EMIT)])
                pltpu.sync_copy(
                    cd.at[pl.ds(i * _EMIT, _EMIT)],
                    cd_all.at[wid, pl.ds(nbt2 * _GB, _EMIT)])
                return nbt2 + _EMIT // _GB
            nbt = lax.fori_loop(0, ne, emit, nbt)
            rem = cnt - ne * _EMIT

            @pl.when(ne > 0)
            def _():
                base = ne * _EMIT
                def mv(k, c):
                    vs = cs[pl.ds(base + k * _L, _L)]
                    vd = cd[pl.ds(base + k * _L, _L)]
                    cs[pl.ds(k * _L, _L)] = vs
                    cd[pl.ds(k * _L, _L)] = vd
                    return c
                lax.fori_loop(0, _EMIT // _L, mv, 0)
            return rem, nbt

        pltpu.async_copy(src.at[pl.ds(0, _CHUNK)], sva, sa)
        pltpu.async_copy(dst.at[pl.ds(0, _CHUNK)], dva, da)

        def pair_body(q, carry):
            rem, nbt = carry
            c = 2 * q
            pltpu.make_async_copy(src.at[pl.ds(c * _CHUNK, _CHUNK)], sva, sa).wait()
            pltpu.make_async_copy(dst.at[pl.ds(c * _CHUNK, _CHUNK)], dva, da).wait()
            pltpu.async_copy(src.at[pl.ds((c + 1) * _CHUNK, _CHUNK)], svb, sb)
            pltpu.async_copy(dst.at[pl.ds((c + 1) * _CHUNK, _CHUNK)], dvb, db)
            cnt = scan_chunk(sva, dva, rem)
            rem, nbt = emit_blocks(cnt, nbt)

            pltpu.make_async_copy(src.at[pl.ds((c + 1) * _CHUNK, _CHUNK)], svb, sb).wait()
            pltpu.make_async_copy(dst.at[pl.ds((c + 1) * _CHUNK, _CHUNK)], dvb, db).wait()

            @pl.when(c + 2 < nchunk)
            def _():
                pltpu.async_copy(src.at[pl.ds((c + 2) * _CHUNK, _CHUNK)], sva, sa)
                pltpu.async_copy(dst.at[pl.ds((c + 2) * _CHUNK, _CHUNK)], dva, da)
            cnt = scan_chunk(svb, dvb, rem)
            return emit_blocks(cnt, nbt)

        rem, nbt = lax.fori_loop(0, nchunk // 2, pair_body, (0, 0))

        zero16i = jnp.zeros((_L,), jnp.int32)
        dummy16 = jnp.full((_L,), rows_per_w, jnp.int32)
        for k in range(_GB // _L):
            cs[pl.ds(rem + k * _L, _L)] = zero16i
            cd[pl.ds(rem + k * _L, _L)] = dummy16

        def flush(b, nbt2):
            pltpu.sync_copy(cs.at[pl.ds(b * _GB, _GB)],
                            cs_all.at[wid, pl.ds(nbt2 * _GB, _GB)])
            pltpu.sync_copy(cd.at[pl.ds(b * _GB, _GB)],
                            cd_all.at[wid, pl.ds(nbt2 * _GB, _GB)])
            return nbt2 + 1
        nbt = lax.fori_loop(0, (rem + _GB - 1) // _GB, flush, nbt)

        cstage[pl.ds(0, _L)] = jnp.full((_L,), nbt, jnp.int32)
        pltpu.sync_copy(cstage, cnts.at[wid])

    return compact


def _build_gather_max(d, rows_per_w, npad, maxb):
    nf = d // _L

    @functools.partial(
        pl.kernel,
        mesh=_mesh(),
        compiler_params=_SC_PARAMS,
        out_type=jax.ShapeDtypeStruct((npad, d), jnp.float32),
        scratch_types=[
            pltpu.VMEM((rows_per_w + 1, d), jnp.float32),
            pltpu.VMEM((_GB,), jnp.int32),
            pltpu.VMEM((_GB,), jnp.int32),
            pltpu.VMEM((_GB,), jnp.int32),
            pltpu.VMEM((_GB,), jnp.int32),
            pltpu.VMEM((_GB, d), jnp.float32),
            pltpu.VMEM((_GB, d), jnp.float32),
            pltpu.VMEM((16,), jnp.int32),
            pltpu.SemaphoreType.DMA,
            pltpu.SemaphoreType.DMA,
        ],
    )
    def gather_max(feat, cs_all, cd_all, cnts, out, acc, cs0, cs1, cd0, cd1,
                   rows0, rows1, cstage, g0, g1):
        wid = lax.axis_index("s") * _NC + lax.axis_index("c")
        lo = wid * rows_per_w
        zero16f = jnp.zeros((_L,), jnp.float32)

        def init_row(r, carry):
            for f in range(nf):
                acc[r, pl.ds(f * _L, _L)] = zero16f
            return carry
        lax.fori_loop(0, rows_per_w + 1, init_row, 0)

        pltpu.sync_copy(cnts.at[wid], cstage)
        nb = cstage[pl.ds(0, _L)][0]

        def maxloop(rows, cdb):
            def grp(g, c3):
                dv16 = cdb[pl.ds(g * _L, _L)]
                for k in range(_L):
                    r = dv16[k]
                    j = g * _L + k
                    for f in range(nf):
                        sl = pl.ds(f * _L, _L)
                        acc[r, sl] = jnp.maximum(acc[r, sl], rows[j, sl])
                return c3
            lax.fori_loop(0, _GB // _L, grp, 0)

        @pl.when(nb > 0)
        def _():
            pltpu.sync_copy(cs_all.at[wid, pl.ds(0, _GB)], cs0)
            pltpu.async_copy(feat.at[cs0], rows0, g0)

        def pair(q, carry):
            b0 = 2 * q
            b1 = b0 + 1

            @pl.when(b1 < nb)
            def _():
                pltpu.sync_copy(cs_all.at[wid, pl.ds(b1 * _GB, _GB)], cs1)
                pltpu.async_copy(feat.at[cs1], rows1, g1)
            pltpu.sync_copy(cd_all.at[wid, pl.ds(b0 * _GB, _GB)], cd0)
            pltpu.make_async_copy(feat.at[cs0], rows0, g0).wait()
            maxloop(rows0, cd0)

            @pl.when(b0 + 2 < nb)
            def _():
                pltpu.sync_copy(cs_all.at[wid, pl.ds((b0 + 2) * _GB, _GB)], cs0)
                pltpu.async_copy(feat.at[cs0], rows0, g0)

            @pl.when(b1 < nb)
            def _():
                pltpu.sync_copy(cd_all.at[wid, pl.ds(b1 * _GB, _GB)], cd1)
                pltpu.make_async_copy(feat.at[cs1], rows1, g1).wait()
                maxloop(rows1, cd1)
            return carry

        lax.fori_loop(0, (nb + 1) // 2, pair, 0)
        pltpu.sync_copy(acc.at[pl.ds(0, rows_per_w)],
                        out.at[pl.ds(lo, rows_per_w)])

    return gather_max


def _edge_lists(src, dst, n_nodes):
    e = src.shape[0]
    rows_per_w = (-(-n_nodes // _NW) + 7) // 8 * 8
    npad = _NW * rows_per_w
    step = 2 * _CHUNK
    epad = -(-e // step) * step
    if epad != e:
        pad = epad - e
        src = jnp.concatenate([src, jnp.zeros((pad,), jnp.int32)])
        dst = jnp.concatenate([dst, jnp.full((pad,), 0x3FFFFFFF, jnp.int32)])
    maxb = epad // _GB + 8
    fn = _build_compact(epad, rows_per_w, maxb)
    cs_all, cd_all, cnts = fn(src, dst)
    return cs_all, cd_all, cnts, rows_per_w, npad, maxb


def _relu_mm(x, w, b, bm):
    n, din = x.shape
    dout = w.shape[1]

    def body(x_ref, w_ref, b_ref, o_ref):
        o_ref[...] = jnp.maximum(
            jnp.dot(x_ref[...], w_ref[...], preferred_element_type=jnp.float32)
            + b_ref[...], 0.0)

    return pl.pallas_call(
        body,
        grid=(n // bm,),
        in_specs=[pl.BlockSpec((bm, din), lambda i: (i, 0)),
                  pl.BlockSpec((din, dout), lambda i: (0, 0)),
                  pl.BlockSpec((1, dout), lambda i: (0, 0))],
        out_specs=pl.BlockSpec((bm, dout), lambda i: (i, 0)),
        out_shape=jax.ShapeDtypeStruct((n, dout), jnp.float32),
    )(x, w, b.reshape(1, -1))


def _ln_relu(t, g_ref, be_ref):
    mu = jnp.mean(t, axis=-1, keepdims=True)
    var = jnp.mean((t - mu) ** 2, axis=-1, keepdims=True)
    hn = (t - mu) / jnp.sqrt(var + 1e-5) * g_ref[...] + be_ref[...]
    return jnp.maximum(hn, 0.0)


def _mid_layer(x, agg, ws, wn, b, g, be, wp1, bp1, bm):
    n, din = x.shape
    h = ws.shape[1]

    def body(x_ref, a_ref, ws_ref, wn_ref, b_ref, g_ref, be_ref, wp_ref,
             bp_ref, h0_ref, m1_ref):
        t = (jnp.dot(x_ref[...], ws_ref[...], preferred_element_type=jnp.float32)
             + jnp.dot(a_ref[...], wn_ref[...], preferred_element_type=jnp.float32)
             + b_ref[...])
        h0 = _ln_relu(t, g_ref, be_ref)
        h0_ref[...] = h0
        m1_ref[...] = jnp.maximum(
            jnp.dot(h0, wp_ref[...], preferred_element_type=jnp.float32)
            + bp_ref[...], 0.0)

    return pl.pallas_call(
        body,
        grid=(n // bm,),
        in_specs=[pl.BlockSpec((bm, din), lambda i: (i, 0)),
                  pl.BlockSpec((bm, din), lambda i: (i, 0)),
                  pl.BlockSpec((din, h), lambda i: (0, 0)),
                  pl.BlockSpec((din, h), lambda i: (0, 0)),
                  pl.BlockSpec((1, h), lambda i: (0, 0)),
                  pl.BlockSpec((1, h), lambda i: (0, 0)),
                  pl.BlockSpec((1, h), lambda i: (0, 0)),
                  pl.BlockSpec((h, h), lambda i: (0, 0)),
                  pl.BlockSpec((1, h), lambda i: (0, 0))],
        out_specs=[pl.BlockSpec((bm, h), lambda i: (i, 0)),
                   pl.BlockSpec((bm, h), lambda i: (i, 0))],
        out_shape=[jax.ShapeDtypeStruct((n, h), jnp.float32),
                   jax.ShapeDtypeStruct((n, h), jnp.float32)],
    )(x, agg, ws, wn, b.reshape(1, -1), g.reshape(1, -1), be.reshape(1, -1),
      wp1, bp1.reshape(1, -1))


def _final_layer(h0, agg, ws, wn, b, g, be, fc1w, fc1b, fcw, fcb, bm):
    n, h = h0.shape
    grid = n // bm

    def body(h0_ref, a_ref, ws_ref, wn_ref, b_ref, g_ref, be_ref, w1_ref,
             b1_ref, w2_ref, b2_ref, o_ref, pooled):
        i = pl.program_id(0)
        t = (jnp.dot(h0_ref[...], ws_ref[...], preferred_element_type=jnp.float32)
             + jnp.dot(a_ref[...], wn_ref[...], preferred_element_type=jnp.float32)
             + b_ref[...])
        h1 = _ln_relu(t, g_ref, be_ref)
        bmax = jnp.max(h1, axis=0, keepdims=True)

        @pl.when(i == 0)
        def _():
            pooled[...] = bmax

        @pl.when(i > 0)
        def _():
            pooled[...] = jnp.maximum(pooled[...], bmax)

        @pl.when(i == grid - 1)
        def _():
            y = (jnp.dot(pooled[...], w1_ref[...],
                         preferred_element_type=jnp.float32) + b1_ref[...])
            o_ref[...] = (jnp.dot(y, w2_ref[...],
                                  preferred_element_type=jnp.float32)
                          + b2_ref[...])

    return pl.pallas_call(
        body,
        grid=(grid,),
        in_specs=[pl.BlockSpec((bm, h), lambda i: (i, 0)),
                  pl.BlockSpec((bm, h), lambda i: (i, 0)),
                  pl.BlockSpec((h, h), lambda i: (0, 0)),
                  pl.BlockSpec((h, h), lambda i: (0, 0)),
                  pl.BlockSpec((1, h), lambda i: (0, 0)),
                  pl.BlockSpec((1, h), lambda i: (0, 0)),
                  pl.BlockSpec((1, h), lambda i: (0, 0)),
                  pl.BlockSpec((h, h), lambda i: (0, 0)),
                  pl.BlockSpec((1, h), lambda i: (0, 0)),
                  pl.BlockSpec((h, 1), lambda i: (0, 0)),
                  pl.BlockSpec((1, 1), lambda i: (0, 0))],
        out_specs=pl.BlockSpec((1, 1), lambda i: (0, 0)),
        out_shape=jax.ShapeDtypeStruct((1, 1), jnp.float32),
        scratch_shapes=[pltpu.VMEM((1, h), jnp.float32)],
    )(h0, agg, ws, wn, b.reshape(1, -1), g.reshape(1, -1), be.reshape(1, -1),
      fc1w, fc1b.reshape(1, -1), fcw, fcb.reshape(1, -1))


def kernel(x, edge_index, Wp0, bp0, Ws0, Wn0, b0, g0, be0, Wp1, bp1, Ws1,
           Wn1, b1, g1, be1, fc1_w, fc1_b, fc_w, fc_b):
    n = x.shape[0]
    src = edge_index[0]
    dst = edge_index[1]
    bm = 1000 if n % 1000 == 0 else n

    cs_all, cd_all, cnts, rows_per_w, npad, maxb = _edge_lists(src, dst, n)
    m0 = _relu_mm(x, Wp0, bp0, bm)
    gm0 = _build_gather_max(x.shape[1], rows_per_w, npad, maxb)
    agg0 = gm0(m0, cs_all, cd_all, cnts)[:n]
    h0, m1 = _mid_layer(x, agg0, Ws0, Wn0, b0, g0, be0, Wp1, bp1, bm)
    gm1 = _build_gather_max(m1.shape[1], rows_per_w, npad, maxb)
    agg1 = gm1(m1, cs_all, cd_all, cnts)[:n]
    return _final_layer(h0, agg1, Ws1, Wn1, b1, g1, be1, fc1_w, fc1_b,
                        fc_w, fc_b, bm)


# speedup vs baseline: 3.5168x; 1.1021x over previous
import functools

import jax
import jax.numpy as jnp
from jax import lax
from jax.experimental import pallas as pl
from jax.experimental.pallas import tpu as pltpu
from jax.experimental.pallas import tpu_sc as plsc

_NC = 2
_NS = 16
_NW = _NC * _NS
_L = 16

_CHUNK = 2000
_GB = 128
_EMIT = 1024
_CB = 3072

_SC_PARAMS = pltpu.CompilerParams(
    needs_layout_passes=False, use_tc_tiling_on_sc=False)


def _mesh():
    return plsc.VectorSubcoreMesh(
        core_axis_name="c", subcore_axis_name="s",
        num_cores=_NC, num_subcores=_NS)


def _build_compact(n_edges, rows_per_w, maxb):
    nchunk = n_edges // _CHUNK
    assert nchunk % 2 == 0

    @functools.partial(
        pl.kernel,
        mesh=_mesh(),
        compiler_params=_SC_PARAMS,
        out_type=(jax.ShapeDtypeStruct((_NW, maxb * _GB), jnp.int32),
                  jax.ShapeDtypeStruct((_NW, maxb * _GB), jnp.int32),
                  jax.ShapeDtypeStruct((_NW, 16), jnp.int32)),
        scratch_types=[
            pltpu.VMEM((_CHUNK,), jnp.int32),
            pltpu.VMEM((_CHUNK,), jnp.int32),
            pltpu.VMEM((_CHUNK,), jnp.int32),
            pltpu.VMEM((_CHUNK,), jnp.int32),
            pltpu.VMEM((_CB,), jnp.int32),
            pltpu.VMEM((_CB,), jnp.int32),
            pltpu.VMEM((16,), jnp.int32),
            pltpu.SemaphoreType.DMA,
            pltpu.SemaphoreType.DMA,
            pltpu.SemaphoreType.DMA,
            pltpu.SemaphoreType.DMA,
        ],
    )
    def compact(src, dst, cs_all, cd_all, cnts, sva, dva, svb, dvb, cs, cd,
                cstage, sa, da, sb, db):
        wid = lax.axis_index("s") * _NC + lax.axis_index("c")
        lo = wid * rows_per_w

        def scan_chunk(sv, dv, cnt):
            def scan(i, cnt2):
                dd = dv[pl.ds(i * _L, _L)]
                ss = sv[pl.ds(i * _L, _L)]
                dl = dd - lo
                m = plsc.bitcast(dl, jnp.uint32) < jnp.uint32(rows_per_w)
                plsc.store_compressed(cs.at[pl.ds(cnt2, _L)], ss, mask=m)
                plsc.store_compressed(cd.at[pl.ds(cnt2, _L)], dl, mask=m)
                pc = plsc.all_reduce_population_count(m)
                return cnt2 + pc[0]
            return lax.fori_loop(0, _CHUNK // _L, scan, cnt)

        def emit_blocks(cnt, nbt):
            ne = cnt // _EMIT

            def emit(i, nbt2):
                pltpu.sync_copy(
                    cs.at[pl.ds(i * _EMIT, _EMIT)],
                    cs_all.at[wid, pl.ds(nbt2 * _GB, _EMIT)])
                pltpu.sync_copy(
                    cd.at[pl.ds(i * _EMIT, _EMIT)],
                    cd_all.at[wid, pl.ds(nbt2 * _GB, _EMIT)])
                return nbt2 + _EMIT // _GB
            nbt = lax.fori_loop(0, ne, emit, nbt)
            rem = cnt - ne * _EMIT

            @pl.when(ne > 0)
            def _():
                base = ne * _EMIT
                def mv(k, c):
                    vs = cs[pl.ds(base + k * _L, _L)]
                    vd = cd[pl.ds(base + k * _L, _L)]
                    cs[pl.ds(k * _L, _L)] = vs
                    cd[pl.ds(k * _L, _L)] = vd
                    return c
                lax.fori_loop(0, _EMIT // _L, mv, 0)
            return rem, nbt

        pltpu.async_copy(src.at[pl.ds(0, _CHUNK)], sva, sa)
        pltpu.async_copy(dst.at[pl.ds(0, _CHUNK)], dva, da)

        def pair_body(q, carry):
            rem, nbt = carry
            c = 2 * q
            pltpu.make_async_copy(src.at[pl.ds(c * _CHUNK, _CHUNK)], sva, sa).wait()
            pltpu.make_async_copy(dst.at[pl.ds(c * _CHUNK, _CHUNK)], dva, da).wait()
            pltpu.async_copy(src.at[pl.ds((c + 1) * _CHUNK, _CHUNK)], svb, sb)
            pltpu.async_copy(dst.at[pl.ds((c + 1) * _CHUNK, _CHUNK)], dvb, db)
            cnt = scan_chunk(sva, dva, rem)
            rem, nbt = emit_blocks(cnt, nbt)

            pltpu.make_async_copy(src.at[pl.ds((c + 1) * _CHUNK, _CHUNK)], svb, sb).wait()
            pltpu.make_async_copy(dst.at[pl.ds((c + 1) * _CHUNK, _CHUNK)], dvb, db).wait()

            @pl.when(c + 2 < nchunk)
            def _():
                pltpu.async_copy(src.at[pl.ds((c + 2) * _CHUNK, _CHUNK)], sva, sa)
                pltpu.async_copy(dst.at[pl.ds((c + 2) * _CHUNK, _CHUNK)], dva, da)
            cnt = scan_chunk(svb, dvb, rem)
            return emit_blocks(cnt, nbt)

        rem, nbt = lax.fori_loop(0, nchunk // 2, pair_body, (0, 0))

        zero16i = jnp.zeros((_L,), jnp.int32)
        dummy16 = jnp.full((_L,), rows_per_w, jnp.int32)
        for k in range(_GB // _L):
            cs[pl.ds(rem + k * _L, _L)] = zero16i
            cd[pl.ds(rem + k * _L, _L)] = dummy16

        def flush(b, nbt2):
            pltpu.sync_copy(cs.at[pl.ds(b * _GB, _GB)],
                            cs_all.at[wid, pl.ds(nbt2 * _GB, _GB)])
            pltpu.sync_copy(cd.at[pl.ds(b * _GB, _GB)],
                            cd_all.at[wid, pl.ds(nbt2 * _GB, _GB)])
            return nbt2 + 1
        nbt = lax.fori_loop(0, (rem + _GB - 1) // _GB, flush, nbt)

        cstage[pl.ds(0, _L)] = jnp.full((_L,), nbt, jnp.int32)
        pltpu.sync_copy(cstage, cnts.at[wid])

    return compact


def _build_gather_max(d, rows_per_w, npad, maxb):
    nf = d // _L
    bpb = _EMIT // _GB

    @functools.partial(
        pl.kernel,
        mesh=_mesh(),
        compiler_params=_SC_PARAMS,
        out_type=jax.ShapeDtypeStruct((npad, d), jnp.float32),
        scratch_types=[
            pltpu.VMEM((rows_per_w + 1, d), jnp.float32),
            pltpu.VMEM((_EMIT,), jnp.int32),
            pltpu.VMEM((_EMIT,), jnp.int32),
            pltpu.VMEM((_EMIT,), jnp.int32),
            pltpu.VMEM((_EMIT,), jnp.int32),
            pltpu.VMEM((_GB, d), jnp.float32),
            pltpu.VMEM((_GB, d), jnp.float32),
            pltpu.VMEM((16,), jnp.int32),
            pltpu.SemaphoreType.DMA,
            pltpu.SemaphoreType.DMA,
            pltpu.SemaphoreType.DMA,
            pltpu.SemaphoreType.DMA,
        ],
    )
    def gather_max(feat, cs_all, cd_all, cnts, out, acc, cs0, cs1, cd0, cd1,
                   rows0, rows1, cstage, g0, g1, i0, i1):
        wid = lax.axis_index("s") * _NC + lax.axis_index("c")
        lo = wid * rows_per_w
        zero16f = jnp.zeros((_L,), jnp.float32)

        def init_row(r, carry):
            for f in range(nf):
                acc[r, pl.ds(f * _L, _L)] = zero16f
            return carry
        lax.fori_loop(0, rows_per_w + 1, init_row, 0)

        pltpu.sync_copy(cnts.at[wid], cstage)
        nb = cstage[pl.ds(0, _L)][0]
        nblk = (nb + bpb - 1) // bpb

        def maxloop(rows, cdb, j):
            def grp(g, c3):
                dv16 = cdb[pl.ds(j * _GB + g * _L, _L)]
                for k in range(_L):
                    r = dv16[k]
                    jj = g * _L + k
                    for f in range(nf):
                        sl = pl.ds(f * _L, _L)
                        acc[r, sl] = jnp.maximum(acc[r, sl], rows[jj, sl])
                return c3
            lax.fori_loop(0, _GB // _L, grp, 0)

        def load_block(n, csb, cdb, sem):
            pltpu.async_copy(cs_all.at[wid, pl.ds(n * _EMIT, _EMIT)], csb, sem)
            pltpu.async_copy(cd_all.at[wid, pl.ds(n * _EMIT, _EMIT)], cdb, sem)

        def wait_block(n, csb, cdb, sem):
            pltpu.make_async_copy(
                cs_all.at[wid, pl.ds(n * _EMIT, _EMIT)], csb, sem).wait()
            pltpu.make_async_copy(
                cd_all.at[wid, pl.ds(n * _EMIT, _EMIT)], cdb, sem).wait()

        def do_block(n, csb, cdb):
            base = n * bpb

            @pl.when(base < nb)
            def _():
                pltpu.async_copy(feat.at[csb.at[pl.ds(0, _GB)]], rows0, g0)
            for j in range(bpb):
                rows, grs = (rows0, g0) if j % 2 == 0 else (rows1, g1)
                nrows, ngs = (rows1, g1) if j % 2 == 0 else (rows0, g0)

                if j + 1 < bpb:
                    @pl.when(base + j + 1 < nb)
                    def _(j=j, nrows=nrows, ngs=ngs):
                        pltpu.async_copy(
                            feat.at[csb.at[pl.ds((j + 1) * _GB, _GB)]],
                            nrows, ngs)

                @pl.when(base + j < nb)
                def _(j=j, rows=rows, grs=grs):
                    pltpu.make_async_copy(
                        feat.at[csb.at[pl.ds(j * _GB, _GB)]], rows, grs).wait()
                    maxloop(rows, cdb, j)

        @pl.when(nblk > 0)
        def _():
            load_block(0, cs0, cd0, i0)

        def blk_pair(q, carry):
            n0 = 2 * q
            n1 = n0 + 1
            wait_block(n0, cs0, cd0, i0)

            @pl.when(n1 < nblk)
            def _():
                load_block(n1, cs1, cd1, i1)
            do_block(n0, cs0, cd0)

            @pl.when(n0 + 2 < nblk)
            def _():
                load_block(n0 + 2, cs0, cd0, i0)

            @pl.when(n1 < nblk)
            def _():
                wait_block(n1, cs1, cd1, i1)
                do_block(n1, cs1, cd1)
            return carry

        lax.fori_loop(0, (nblk + 1) // 2, blk_pair, 0)
        pltpu.sync_copy(acc.at[pl.ds(0, rows_per_w)],
                        out.at[pl.ds(lo, rows_per_w)])

    return gather_max


def _edge_lists(src, dst, n_nodes):
    e = src.shape[0]
    rows_per_w = (-(-n_nodes // _NW) + 7) // 8 * 8
    npad = _NW * rows_per_w
    step = 2 * _CHUNK
    epad = -(-e // step) * step
    if epad != e:
        pad = epad - e
        src = jnp.concatenate([src, jnp.zeros((pad,), jnp.int32)])
        dst = jnp.concatenate([dst, jnp.full((pad,), 0x3FFFFFFF, jnp.int32)])
    maxb = epad // _GB + 8
    fn = _build_compact(epad, rows_per_w, maxb)
    cs_all, cd_all, cnts = fn(src, dst)
    return cs_all, cd_all, cnts, rows_per_w, npad, maxb


def _relu_mm(x, w, b, bm):
    n, din = x.shape
    dout = w.shape[1]

    def body(x_ref, w_ref, b_ref, o_ref):
        o_ref[...] = jnp.maximum(
            jnp.dot(x_ref[...], w_ref[...], preferred_element_type=jnp.float32)
            + b_ref[...], 0.0)

    return pl.pallas_call(
        body,
        grid=(n // bm,),
        in_specs=[pl.BlockSpec((bm, din), lambda i: (i, 0)),
                  pl.BlockSpec((din, dout), lambda i: (0, 0)),
                  pl.BlockSpec((1, dout), lambda i: (0, 0))],
        out_specs=pl.BlockSpec((bm, dout), lambda i: (i, 0)),
        out_shape=jax.ShapeDtypeStruct((n, dout), jnp.float32),
    )(x, w, b.reshape(1, -1))


def _ln_relu(t, g_ref, be_ref):
    mu = jnp.mean(t, axis=-1, keepdims=True)
    var = jnp.mean((t - mu) ** 2, axis=-1, keepdims=True)
    hn = (t - mu) / jnp.sqrt(var + 1e-5) * g_ref[...] + be_ref[...]
    return jnp.maximum(hn, 0.0)


def _mid_layer(x, agg, ws, wn, b, g, be, wp1, bp1, bm):
    n, din = x.shape
    h = ws.shape[1]

    def body(x_ref, a_ref, ws_ref, wn_ref, b_ref, g_ref, be_ref, wp_ref,
             bp_ref, h0_ref, m1_ref):
        t = (jnp.dot(x_ref[...], ws_ref[...], preferred_element_type=jnp.float32)
             + jnp.dot(a_ref[...], wn_ref[...], preferred_element_type=jnp.float32)
             + b_ref[...])
        h0 = _ln_relu(t, g_ref, be_ref)
        h0_ref[...] = h0
        m1_ref[...] = jnp.maximum(
            jnp.dot(h0, wp_ref[...], preferred_element_type=jnp.float32)
            + bp_ref[...], 0.0)

    return pl.pallas_call(
        body,
        grid=(n // bm,),
        in_specs=[pl.BlockSpec((bm, din), lambda i: (i, 0)),
                  pl.BlockSpec((bm, din), lambda i: (i, 0)),
                  pl.BlockSpec((din, h), lambda i: (0, 0)),
                  pl.BlockSpec((din, h), lambda i: (0, 0)),
                  pl.BlockSpec((1, h), lambda i: (0, 0)),
                  pl.BlockSpec((1, h), lambda i: (0, 0)),
                  pl.BlockSpec((1, h), lambda i: (0, 0)),
                  pl.BlockSpec((h, h), lambda i: (0, 0)),
                  pl.BlockSpec((1, h), lambda i: (0, 0))],
        out_specs=[pl.BlockSpec((bm, h), lambda i: (i, 0)),
                   pl.BlockSpec((bm, h), lambda i: (i, 0))],
        out_shape=[jax.ShapeDtypeStruct((n, h), jnp.float32),
                   jax.ShapeDtypeStruct((n, h), jnp.float32)],
    )(x, agg, ws, wn, b.reshape(1, -1), g.reshape(1, -1), be.reshape(1, -1),
      wp1, bp1.reshape(1, -1))


def _final_layer(h0, agg, ws, wn, b, g, be, fc1w, fc1b, fcw, fcb, bm):
    n, h = h0.shape
    grid = n // bm

    def body(h0_ref, a_ref, ws_ref, wn_ref, b_ref, g_ref, be_ref, w1_ref,
             b1_ref, w2_ref, b2_ref, o_ref, pooled):
        i = pl.program_id(0)
        t = (jnp.dot(h0_ref[...], ws_ref[...], preferred_element_type=jnp.float32)
             + jnp.dot(a_ref[...], wn_ref[...], preferred_element_type=jnp.float32)
             + b_ref[...])
        h1 = _ln_relu(t, g_ref, be_ref)
        bmax = jnp.max(h1, axis=0, keepdims=True)

        @pl.when(i == 0)
        def _():
            pooled[...] = bmax

        @pl.when(i > 0)
        def _():
            pooled[...] = jnp.maximum(pooled[...], bmax)

        @pl.when(i == grid - 1)
        def _():
            y = (jnp.dot(pooled[...], w1_ref[...],
                         preferred_element_type=jnp.float32) + b1_ref[...])
            o_ref[...] = (jnp.dot(y, w2_ref[...],
                                  preferred_element_type=jnp.float32)
                          + b2_ref[...])

    return pl.pallas_call(
        body,
        grid=(grid,),
        in_specs=[pl.BlockSpec((bm, h), lambda i: (i, 0)),
                  pl.BlockSpec((bm, h), lambda i: (i, 0)),
                  pl.BlockSpec((h, h), lambda i: (0, 0)),
                  pl.BlockSpec((h, h), lambda i: (0, 0)),
                  pl.BlockSpec((1, h), lambda i: (0, 0)),
                  pl.BlockSpec((1, h), lambda i: (0, 0)),
                  pl.BlockSpec((1, h), lambda i: (0, 0)),
                  pl.BlockSpec((h, h), lambda i: (0, 0)),
                  pl.BlockSpec((1, h), lambda i: (0, 0)),
                  pl.BlockSpec((h, 1), lambda i: (0, 0)),
                  pl.BlockSpec((1, 1), lambda i: (0, 0))],
        out_specs=pl.BlockSpec((1, 1), lambda i: (0, 0)),
        out_shape=jax.ShapeDtypeStruct((1, 1), jnp.float32),
        scratch_shapes=[pltpu.VMEM((1, h), jnp.float32)],
    )(h0, agg, ws, wn, b.reshape(1, -1), g.reshape(1, -1), be.reshape(1, -1),
      fc1w, fc1b.reshape(1, -1), fcw, fcb.reshape(1, -1))


def kernel(x, edge_index, Wp0, bp0, Ws0, Wn0, b0, g0, be0, Wp1, bp1, Ws1,
           Wn1, b1, g1, be1, fc1_w, fc1_b, fc_w, fc_b):
    n = x.shape[0]
    src = edge_index[0]
    dst = edge_index[1]
    bm = 1000 if n % 1000 == 0 else n

    cs_all, cd_all, cnts, rows_per_w, npad, maxb = _edge_lists(src, dst, n)
    m0 = _relu_mm(x, Wp0, bp0, bm)
    gm0 = _build_gather_max(x.shape[1], rows_per_w, npad, maxb)
    agg0 = gm0(m0, cs_all, cd_all, cnts)[:n]
    h0, m1 = _mid_layer(x, agg0, Ws0, Wn0, b0, g0, be0, Wp1, bp1, bm)
    gm1 = _build_gather_max(m1.shape[1], rows_per_w, npad, maxb)
    agg1 = gm1(m1, cs_all, cd_all, cnts)[:n]
    return _final_layer(h0, agg1, Ws1, Wn1, b1, g1, be1, fc1_w, fc1_b,
                        fc_w, fc_b, bm)


# speedup vs baseline: 3.8789x; 1.1030x over previous
import functools

import jax
import jax.numpy as jnp
from jax import lax
from jax.experimental import pallas as pl
from jax.experimental.pallas import tpu as pltpu
from jax.experimental.pallas import tpu_sc as plsc

_NC = 2
_NS = 16
_NW = _NC * _NS
_L = 16

_CHUNK = 1600
_GB = 128
_EMIT = 1024
_CB = 3072

_SC_PARAMS = pltpu.CompilerParams(
    needs_layout_passes=False, use_tc_tiling_on_sc=False)


def _mesh():
    return plsc.VectorSubcoreMesh(
        core_axis_name="c", subcore_axis_name="s",
        num_cores=_NC, num_subcores=_NS)


def _build_compact(n_edges, rows_per_w, maxb):
    nchunk = n_edges // _CHUNK
    assert nchunk % 2 == 0

    @functools.partial(
        pl.kernel,
        mesh=_mesh(),
        compiler_params=_SC_PARAMS,
        out_type=(jax.ShapeDtypeStruct((_NW, maxb * _GB), jnp.int32),
                  jax.ShapeDtypeStruct((_NW, maxb * _GB), jnp.int32),
                  jax.ShapeDtypeStruct((_NW, 16), jnp.int32)),
        scratch_types=[
            pltpu.VMEM((_CHUNK,), jnp.int32),
            pltpu.VMEM((_CHUNK,), jnp.int32),
            pltpu.VMEM((_CHUNK,), jnp.int32),
            pltpu.VMEM((_CHUNK,), jnp.int32),
            pltpu.VMEM((_CB,), jnp.int32),
            pltpu.VMEM((_CB,), jnp.int32),
            pltpu.VMEM((16,), jnp.int32),
            pltpu.SemaphoreType.DMA,
            pltpu.SemaphoreType.DMA,
            pltpu.SemaphoreType.DMA,
            pltpu.SemaphoreType.DMA,
        ],
    )
    def compact(src, dst, cs_all, cd_all, cnts, sva, dva, svb, dvb, cs, cd,
                cstage, sa, da, sb, db):
        wid = lax.axis_index("s") * _NC + lax.axis_index("c")
        lo = wid * rows_per_w

        def scan_chunk(sv, dv, cnt):
            def scan(i, cnt2):
                dls, sss, ms, pcs = [], [], [], []
                for u in range(4):
                    dd = dv[pl.ds((i * 4 + u) * _L, _L)]
                    ss = sv[pl.ds((i * 4 + u) * _L, _L)]
                    dl = dd - lo
                    m = plsc.bitcast(dl, jnp.uint32) < jnp.uint32(rows_per_w)
                    dls.append(dl)
                    sss.append(ss)
                    ms.append(m)
                    pcs.append(plsc.all_reduce_population_count(m)[0])
                for u in range(4):
                    plsc.store_compressed(cs.at[pl.ds(cnt2, _L)], sss[u],
                                          mask=ms[u])
                    plsc.store_compressed(cd.at[pl.ds(cnt2, _L)], dls[u],
                                          mask=ms[u])
                    cnt2 = cnt2 + pcs[u]
                return cnt2
            return lax.fori_loop(0, _CHUNK // (4 * _L), scan, cnt)

        def emit_blocks(cnt, nbt):
            ne = cnt // _EMIT

            def emit(i, nbt2):
                pltpu.sync_copy(
                    cs.at[pl.ds(i * _EMIT, _EMIT)],
                    cs_all.at[wid, pl.ds(nbt2 * _GB, _EMIT)])
                pltpu.sync_copy(
                    cd.at[pl.ds(i * _EMIT, _EMIT)],
                    cd_all.at[wid, pl.ds(nbt2 * _GB, _EMIT)])
                return nbt2 + _EMIT // _GB
            nbt = lax.fori_loop(0, ne, emit, nbt)
            rem = cnt - ne * _EMIT

            @pl.when(ne > 0)
            def _():
                base = ne * _EMIT
                def mv(k, c):
                    vs = cs[pl.ds(base + k * _L, _L)]
                    vd = cd[pl.ds(base + k * _L, _L)]
                    cs[pl.ds(k * _L, _L)] = vs
                    cd[pl.ds(k * _L, _L)] = vd
                    return c
                lax.fori_loop(0, _EMIT // _L, mv, 0)
            return rem, nbt

        pltpu.async_copy(src.at[pl.ds(0, _CHUNK)], sva, sa)
        pltpu.async_copy(dst.at[pl.ds(0, _CHUNK)], dva, da)

        def pair_body(q, carry):
            rem, nbt = carry
            c = 2 * q
            pltpu.make_async_copy(src.at[pl.ds(c * _CHUNK, _CHUNK)], sva, sa).wait()
            pltpu.make_async_copy(dst.at[pl.ds(c * _CHUNK, _CHUNK)], dva, da).wait()
            pltpu.async_copy(src.at[pl.ds((c + 1) * _CHUNK, _CHUNK)], svb, sb)
            pltpu.async_copy(dst.at[pl.ds((c + 1) * _CHUNK, _CHUNK)], dvb, db)
            cnt = scan_chunk(sva, dva, rem)
            rem, nbt = emit_blocks(cnt, nbt)

            pltpu.make_async_copy(src.at[pl.ds((c + 1) * _CHUNK, _CHUNK)], svb, sb).wait()
            pltpu.make_async_copy(dst.at[pl.ds((c + 1) * _CHUNK, _CHUNK)], dvb, db).wait()

            @pl.when(c + 2 < nchunk)
            def _():
                pltpu.async_copy(src.at[pl.ds((c + 2) * _CHUNK, _CHUNK)], sva, sa)
                pltpu.async_copy(dst.at[pl.ds((c + 2) * _CHUNK, _CHUNK)], dva, da)
            cnt = scan_chunk(svb, dvb, rem)
            return emit_blocks(cnt, nbt)

        rem, nbt = lax.fori_loop(0, nchunk // 2, pair_body, (0, 0))

        zero16i = jnp.zeros((_L,), jnp.int32)
        dummy16 = jnp.full((_L,), rows_per_w, jnp.int32)
        for k in range(_GB // _L):
            cs[pl.ds(rem + k * _L, _L)] = zero16i
            cd[pl.ds(rem + k * _L, _L)] = dummy16

        def flush(b, nbt2):
            pltpu.sync_copy(cs.at[pl.ds(b * _GB, _GB)],
                            cs_all.at[wid, pl.ds(nbt2 * _GB, _GB)])
            pltpu.sync_copy(cd.at[pl.ds(b * _GB, _GB)],
                            cd_all.at[wid, pl.ds(nbt2 * _GB, _GB)])
            return nbt2 + 1
        nbt = lax.fori_loop(0, (rem + _GB - 1) // _GB, flush, nbt)

        cstage[pl.ds(0, _L)] = jnp.full((_L,), nbt, jnp.int32)
        pltpu.sync_copy(cstage, cnts.at[wid])

    return compact


def _build_gather_max(d, rows_per_w, npad, maxb):
    nf = d // _L
    bpb = _EMIT // _GB

    @functools.partial(
        pl.kernel,
        mesh=_mesh(),
        compiler_params=_SC_PARAMS,
        out_type=jax.ShapeDtypeStruct((npad, d), jnp.float32),
        scratch_types=[
            pltpu.VMEM((rows_per_w + 1, d), jnp.float32),
            pltpu.VMEM((_EMIT,), jnp.int32),
            pltpu.VMEM((_EMIT,), jnp.int32),
            pltpu.VMEM((_EMIT,), jnp.int32),
            pltpu.VMEM((_EMIT,), jnp.int32),
            pltpu.VMEM((_GB, d), jnp.float32),
            pltpu.VMEM((_GB, d), jnp.float32),
            pltpu.VMEM((_GB, d), jnp.float32),
            pltpu.VMEM((_GB, d), jnp.float32),
            pltpu.VMEM((16,), jnp.int32),
            pltpu.SemaphoreType.DMA,
            pltpu.SemaphoreType.DMA,
            pltpu.SemaphoreType.DMA,
            pltpu.SemaphoreType.DMA,
            pltpu.SemaphoreType.DMA,
            pltpu.SemaphoreType.DMA,
        ],
    )
    def gather_max(feat, cs_all, cd_all, cnts, out, acc, cs0, cs1, cd0, cd1,
                   rows0, rows1, rows2, rows3, cstage, g0, g1, g2, g3, i0, i1):
        wid = lax.axis_index("s") * _NC + lax.axis_index("c")
        lo = wid * rows_per_w
        zero16f = jnp.zeros((_L,), jnp.float32)

        def init_row(r, carry):
            for f in range(nf):
                acc[r, pl.ds(f * _L, _L)] = zero16f
            return carry
        lax.fori_loop(0, rows_per_w + 1, init_row, 0)

        pltpu.sync_copy(cnts.at[wid], cstage)
        nb = cstage[pl.ds(0, _L)][0]
        nblk = (nb + bpb - 1) // bpb

        def maxloop(rows, cdb, j):
            def grp(g, c3):
                dv16 = cdb[pl.ds(j * _GB + g * _L, _L)]
                for k in range(_L):
                    r = dv16[k]
                    jj = g * _L + k
                    for f in range(nf):
                        sl = pl.ds(f * _L, _L)
                        acc[r, sl] = jnp.maximum(acc[r, sl], rows[jj, sl])
                return c3
            lax.fori_loop(0, _GB // _L, grp, 0)

        def load_block(n, csb, cdb, sem):
            pltpu.async_copy(cs_all.at[wid, pl.ds(n * _EMIT, _EMIT)], csb, sem)
            pltpu.async_copy(cd_all.at[wid, pl.ds(n * _EMIT, _EMIT)], cdb, sem)

        def wait_block(n, csb, cdb, sem):
            pltpu.make_async_copy(
                cs_all.at[wid, pl.ds(n * _EMIT, _EMIT)], csb, sem).wait()
            pltpu.make_async_copy(
                cd_all.at[wid, pl.ds(n * _EMIT, _EMIT)], cdb, sem).wait()

        ring = [(rows0, g0), (rows1, g1), (rows2, g2), (rows3, g3)]

        def do_block(n, csb, cdb):
            base = n * bpb
            for j in range(3):
                rows, grs = ring[j]

                @pl.when(base + j < nb)
                def _(j=j, rows=rows, grs=grs):
                    pltpu.async_copy(
                        feat.at[csb.at[pl.ds(j * _GB, _GB)]], rows, grs)
            for j in range(bpb):
                rows, grs = ring[j % 4]
                if j + 3 < bpb:
                    nrows, ngs = ring[(j + 3) % 4]

                    @pl.when(base + j + 3 < nb)
                    def _(j=j, nrows=nrows, ngs=ngs):
                        pltpu.async_copy(
                            feat.at[csb.at[pl.ds((j + 3) * _GB, _GB)]],
                            nrows, ngs)

                @pl.when(base + j < nb)
                def _(j=j, rows=rows, grs=grs):
                    pltpu.make_async_copy(
                        feat.at[csb.at[pl.ds(j * _GB, _GB)]], rows, grs).wait()
                    maxloop(rows, cdb, j)

        @pl.when(nblk > 0)
        def _():
            load_block(0, cs0, cd0, i0)

        def blk_pair(q, carry):
            n0 = 2 * q
            n1 = n0 + 1
            wait_block(n0, cs0, cd0, i0)

            @pl.when(n1 < nblk)
            def _():
                load_block(n1, cs1, cd1, i1)
            do_block(n0, cs0, cd0)

            @pl.when(n0 + 2 < nblk)
            def _():
                load_block(n0 + 2, cs0, cd0, i0)

            @pl.when(n1 < nblk)
            def _():
                wait_block(n1, cs1, cd1, i1)
                do_block(n1, cs1, cd1)
            return carry

        lax.fori_loop(0, (nblk + 1) // 2, blk_pair, 0)
        pltpu.sync_copy(acc.at[pl.ds(0, rows_per_w)],
                        out.at[pl.ds(lo, rows_per_w)])

    return gather_max


def _edge_lists(src, dst, n_nodes):
    e = src.shape[0]
    rows_per_w = (-(-n_nodes // _NW) + 7) // 8 * 8
    npad = _NW * rows_per_w
    step = 2 * _CHUNK
    epad = -(-e // step) * step
    if epad != e:
        pad = epad - e
        src = jnp.concatenate([src, jnp.zeros((pad,), jnp.int32)])
        dst = jnp.concatenate([dst, jnp.full((pad,), 0x3FFFFFFF, jnp.int32)])
    maxb = epad // _GB + 8
    fn = _build_compact(epad, rows_per_w, maxb)
    cs_all, cd_all, cnts = fn(src, dst)
    return cs_all, cd_all, cnts, rows_per_w, npad, maxb


def _relu_mm(x, w, b, bm):
    n, din = x.shape
    dout = w.shape[1]

    def body(x_ref, w_ref, b_ref, o_ref):
        o_ref[...] = jnp.maximum(
            jnp.dot(x_ref[...], w_ref[...], preferred_element_type=jnp.float32)
            + b_ref[...], 0.0)

    return pl.pallas_call(
        body,
        grid=(n // bm,),
        in_specs=[pl.BlockSpec((bm, din), lambda i: (i, 0)),
                  pl.BlockSpec((din, dout), lambda i: (0, 0)),
                  pl.BlockSpec((1, dout), lambda i: (0, 0))],
        out_specs=pl.BlockSpec((bm, dout), lambda i: (i, 0)),
        out_shape=jax.ShapeDtypeStruct((n, dout), jnp.float32),
    )(x, w, b.reshape(1, -1))


def _ln_relu(t, g_ref, be_ref):
    mu = jnp.mean(t, axis=-1, keepdims=True)
    var = jnp.mean((t - mu) ** 2, axis=-1, keepdims=True)
    hn = (t - mu) / jnp.sqrt(var + 1e-5) * g_ref[...] + be_ref[...]
    return jnp.maximum(hn, 0.0)


def _mid_layer(x, agg, ws, wn, b, g, be, wp1, bp1, bm):
    n, din = x.shape
    h = ws.shape[1]

    def body(x_ref, a_ref, ws_ref, wn_ref, b_ref, g_ref, be_ref, wp_ref,
             bp_ref, h0_ref, m1_ref):
        t = (jnp.dot(x_ref[...], ws_ref[...], preferred_element_type=jnp.float32)
             + jnp.dot(a_ref[...], wn_ref[...], preferred_element_type=jnp.float32)
             + b_ref[...])
        h0 = _ln_relu(t, g_ref, be_ref)
        h0_ref[...] = h0
        m1_ref[...] = jnp.maximum(
            jnp.dot(h0, wp_ref[...], preferred_element_type=jnp.float32)
            + bp_ref[...], 0.0)

    return pl.pallas_call(
        body,
        grid=(n // bm,),
        in_specs=[pl.BlockSpec((bm, din), lambda i: (i, 0)),
                  pl.BlockSpec((bm, din), lambda i: (i, 0)),
                  pl.BlockSpec((din, h), lambda i: (0, 0)),
                  pl.BlockSpec((din, h), lambda i: (0, 0)),
                  pl.BlockSpec((1, h), lambda i: (0, 0)),
                  pl.BlockSpec((1, h), lambda i: (0, 0)),
                  pl.BlockSpec((1, h), lambda i: (0, 0)),
                  pl.BlockSpec((h, h), lambda i: (0, 0)),
                  pl.BlockSpec((1, h), lambda i: (0, 0))],
        out_specs=[pl.BlockSpec((bm, h), lambda i: (i, 0)),
                   pl.BlockSpec((bm, h), lambda i: (i, 0))],
        out_shape=[jax.ShapeDtypeStruct((n, h), jnp.float32),
                   jax.ShapeDtypeStruct((n, h), jnp.float32)],
    )(x, agg, ws, wn, b.reshape(1, -1), g.reshape(1, -1), be.reshape(1, -1),
      wp1, bp1.reshape(1, -1))


def _final_layer(h0, agg, ws, wn, b, g, be, fc1w, fc1b, fcw, fcb, bm):
    n, h = h0.shape
    grid = n // bm

    def body(h0_ref, a_ref, ws_ref, wn_ref, b_ref, g_ref, be_ref, w1_ref,
             b1_ref, w2_ref, b2_ref, o_ref, pooled):
        i = pl.program_id(0)
        t = (jnp.dot(h0_ref[...], ws_ref[...], preferred_element_type=jnp.float32)
             + jnp.dot(a_ref[...], wn_ref[...], preferred_element_type=jnp.float32)
             + b_ref[...])
        h1 = _ln_relu(t, g_ref, be_ref)
        bmax = jnp.max(h1, axis=0, keepdims=True)

        @pl.when(i == 0)
        def _():
            pooled[...] = bmax

        @pl.when(i > 0)
        def _():
            pooled[...] = jnp.maximum(pooled[...], bmax)

        @pl.when(i == grid - 1)
        def _():
            y = (jnp.dot(pooled[...], w1_ref[...],
                         preferred_element_type=jnp.float32) + b1_ref[...])
            o_ref[...] = (jnp.dot(y, w2_ref[...],
                                  preferred_element_type=jnp.float32)
                          + b2_ref[...])

    return pl.pallas_call(
        body,
        grid=(grid,),
        in_specs=[pl.BlockSpec((bm, h), lambda i: (i, 0)),
                  pl.BlockSpec((bm, h), lambda i: (i, 0)),
                  pl.BlockSpec((h, h), lambda i: (0, 0)),
                  pl.BlockSpec((h, h), lambda i: (0, 0)),
                  pl.BlockSpec((1, h), lambda i: (0, 0)),
                  pl.BlockSpec((1, h), lambda i: (0, 0)),
                  pl.BlockSpec((1, h), lambda i: (0, 0)),
                  pl.BlockSpec((h, h), lambda i: (0, 0)),
                  pl.BlockSpec((1, h), lambda i: (0, 0)),
                  pl.BlockSpec((h, 1), lambda i: (0, 0)),
                  pl.BlockSpec((1, 1), lambda i: (0, 0))],
        out_specs=pl.BlockSpec((1, 1), lambda i: (0, 0)),
        out_shape=jax.ShapeDtypeStruct((1, 1), jnp.float32),
        scratch_shapes=[pltpu.VMEM((1, h), jnp.float32)],
    )(h0, agg, ws, wn, b.reshape(1, -1), g.reshape(1, -1), be.reshape(1, -1),
      fc1w, fc1b.reshape(1, -1), fcw, fcb.reshape(1, -1))


def kernel(x, edge_index, Wp0, bp0, Ws0, Wn0, b0, g0, be0, Wp1, bp1, Ws1,
           Wn1, b1, g1, be1, fc1_w, fc1_b, fc_w, fc_b):
    n = x.shape[0]
    src = edge_index[0]
    dst = edge_index[1]
    bm = 1000 if n % 1000 == 0 else n

    cs_all, cd_all, cnts, rows_per_w, npad, maxb = _edge_lists(src, dst, n)
    m0 = _relu_mm(x, Wp0, bp0, bm)
    gm0 = _build_gather_max(x.shape[1], rows_per_w, npad, maxb)
    agg0 = gm0(m0, cs_all, cd_all, cnts)[:n]
    h0, m1 = _mid_layer(x, agg0, Ws0, Wn0, b0, g0, be0, Wp1, bp1, bm)
    gm1 = _build_gather_max(m1.shape[1], rows_per_w, npad, maxb)
    agg1 = gm1(m1, cs_all, cd_all, cnts)[:n]
    return _final_layer(h0, agg1, Ws1, Wn1, b1, g1, be1, fc1_w, fc1_b,
                        fc_w, fc_b, bm)
